# baseline (device time: 32695 ns/iter reference)
import jax
import jax.numpy as jnp
from jax import lax
from jax.experimental import pallas as pl
from jax.experimental.pallas import tpu as pltpu

N_DEV = 16
B = 128
D = 128
RPS = B // N_DEV


def kernel(x, Win0, Wout0, Win1, Wout1, Win2, Wout2):
    def body(x_ref, win0, wout0, win1, wout1, win2, wout2,
             out_ref, acc_ref, comm_ref, rs_ref,
             send_sems, recv_sems, rs_send_sems, rs_recv_sems):
        my = lax.axis_index("i")
        j = my % 4
        k = my // 4
        base = my - j

        wins = [win0, win1, win2]
        wouts = [wout0, wout1, wout2]

        barrier_sem = pltpu.get_barrier_semaphore()
        for d in (1, 2, 3):
            for peer in (base + (j + d) % 4, 4 * ((k + d) % 4) + j):
                pl.semaphore_signal(
                    barrier_sem, inc=1,
                    device_id=(peer,), device_id_type=pl.DeviceIdType.MESH,
                )

        def exchange(r, phase, peer_of):
            rdmas = []
            for d in (1, 2, 3):
                rdma = pltpu.make_async_remote_copy(
                    src_ref=acc_ref,
                    dst_ref=comm_ref.at[r, phase, d - 1],
                    send_sem=send_sems.at[r, phase, d - 1],
                    recv_sem=recv_sems.at[r, phase, d - 1],
                    device_id=(peer_of(d),),
                    device_id_type=pl.DeviceIdType.MESH,
                )
                rdma.start()
                rdmas.append(rdma)
            for rdma in rdmas:
                rdma.wait()
            acc_ref[:, :] = (
                acc_ref[:, :]
                + comm_ref[r, phase, 0]
                + comm_ref[r, phase, 1]
                + comm_ref[r, phase, 2]
            )

        for r in range(3):
            xcur = x_ref[:, :] if r == 0 else acc_ref[:, :]
            h = jnp.maximum(
                jnp.dot(xcur, wins[r][:, :], preferred_element_type=jnp.float32),
                0.0,
            )
            acc_ref[:, :] = jnp.dot(
                h, wouts[r][:, :], preferred_element_type=jnp.float32
            )

            if r == 0:
                pl.semaphore_wait(barrier_sem, 6)

            if r < 2:
                exchange(r, 0, lambda d: base + (j + d) % 4)
                exchange(r, 1, lambda d: 4 * ((k + d) % 4) + j)
            else:
                rdmas = []
                for d in range(1, N_DEV):
                    tgt = (my + d) % N_DEV
                    rdma = pltpu.make_async_remote_copy(
                        src_ref=acc_ref.at[pl.ds(tgt * RPS, RPS), :],
                        dst_ref=rs_ref.at[d - 1],
                        send_sem=rs_send_sems.at[d - 1],
                        recv_sem=rs_recv_sems.at[d - 1],
                        device_id=(tgt,),
                        device_id_type=pl.DeviceIdType.MESH,
                    )
                    rdma.start()
                    rdmas.append(rdma)
                for rdma in rdmas:
                    rdma.wait()
                total = acc_ref[pl.ds(my * RPS, RPS), :]
                for d in range(1, N_DEV):
                    total = total + rs_ref[d - 1]
                out_ref[:, :] = total

    return pl.pallas_call(
        body,
        out_shape=jax.ShapeDtypeStruct((RPS, D), jnp.float32),
        in_specs=[pl.BlockSpec(memory_space=pltpu.VMEM)] * 7,
        out_specs=pl.BlockSpec(memory_space=pltpu.VMEM),
        scratch_shapes=[
            pltpu.VMEM((B, D), jnp.float32),
            pltpu.VMEM((3, 2, 3, B, D), jnp.float32),
            pltpu.VMEM((N_DEV - 1, RPS, D), jnp.float32),
            pltpu.SemaphoreType.DMA((3, 2, 3)),
            pltpu.SemaphoreType.DMA((3, 2, 3)),
            pltpu.SemaphoreType.DMA((N_DEV - 1,)),
            pltpu.SemaphoreType.DMA((N_DEV - 1,)),
        ],
        compiler_params=pltpu.CompilerParams(collective_id=0),
    )(x, Win0, Wout0, Win1, Wout1, Win2, Wout2)


# device time: 30469 ns/iter; 1.0731x vs baseline; 1.0731x over previous
import jax
import jax.numpy as jnp
from jax import lax
from jax.experimental import pallas as pl
from jax.experimental.pallas import tpu as pltpu

N_DEV = 16
B = 128
D = 128
RPS = B // N_DEV
S = 2
RH = B // S


def kernel(x, Win0, Wout0, Win1, Wout1, Win2, Wout2):
    def body(x_ref, win0, wout0, win1, wout1, win2, wout2,
             out_ref, acc_ref, comm_ref, rs_ref,
             send_sems, recv_sems, rs_send_sems, rs_recv_sems):
        my = lax.axis_index("i")
        j = my % 4
        k = my // 4
        base = my - j

        wins = [win0, win1, win2]
        wouts = [wout0, wout1, wout2]

        barrier_sem = pltpu.get_barrier_semaphore()
        for d in (1, 2, 3):
            for peer in (base + (j + d) % 4, 4 * ((k + d) % 4) + j):
                pl.semaphore_signal(
                    barrier_sem, inc=1,
                    device_id=(peer,), device_id_type=pl.DeviceIdType.MESH,
                )

        peers = [
            lambda d: base + (j + d) % 4,
            lambda d: 4 * ((k + d) % 4) + j,
        ]
        handles = {}

        def rows(s):
            return pl.ds(s * RH, RH)

        def start_phase(r, p, s):
            lst = []
            for d in (1, 2, 3):
                rdma = pltpu.make_async_remote_copy(
                    src_ref=acc_ref.at[rows(s), :],
                    dst_ref=comm_ref.at[r, p, d - 1, s],
                    send_sem=send_sems.at[r, p, d - 1, s],
                    recv_sem=recv_sems.at[r, p, d - 1, s],
                    device_id=(peers[p](d),),
                    device_id_type=pl.DeviceIdType.MESH,
                )
                rdma.start()
                lst.append(rdma)
            handles[(r, p, s)] = lst

        def finish_phase(r, p, s):
            for rdma in handles[(r, p, s)]:
                rdma.wait()
            acc_ref[rows(s), :] = (
                acc_ref[rows(s), :]
                + comm_ref[r, p, 0, s]
                + comm_ref[r, p, 1, s]
                + comm_ref[r, p, 2, s]
            )

        def compute(r, s):
            xcur = x_ref[rows(s), :] if r == 0 else acc_ref[rows(s), :]
            h = jnp.maximum(
                jnp.dot(xcur, wins[r][:, :], preferred_element_type=jnp.float32),
                0.0,
            )
            acc_ref[rows(s), :] = jnp.dot(
                h, wouts[r][:, :], preferred_element_type=jnp.float32
            )

        compute(0, 0)
        pl.semaphore_wait(barrier_sem, 6)
        start_phase(0, 0, 0)
        for s in range(1, S):
            compute(0, s)
            start_phase(0, 0, s)

        for r in (0, 1):
            for s in range(S):
                finish_phase(r, 0, s)
                start_phase(r, 1, s)
            for s in range(S):
                finish_phase(r, 1, s)
                compute(r + 1, s)
                if r == 0:
                    start_phase(1, 0, s)

        rdmas = []
        for d in range(1, N_DEV):
            tgt = (my + d) % N_DEV
            rdma = pltpu.make_async_remote_copy(
                src_ref=acc_ref.at[pl.ds(tgt * RPS, RPS), :],
                dst_ref=rs_ref.at[d - 1],
                send_sem=rs_send_sems.at[d - 1],
                recv_sem=rs_recv_sems.at[d - 1],
                device_id=(tgt,),
                device_id_type=pl.DeviceIdType.MESH,
            )
            rdma.start()
            rdmas.append(rdma)
        for rdma in rdmas:
            rdma.wait()
        total = acc_ref[pl.ds(my * RPS, RPS), :]
        for d in range(1, N_DEV):
            total = total + rs_ref[d - 1]
        out_ref[:, :] = total

    return pl.pallas_call(
        body,
        out_shape=jax.ShapeDtypeStruct((RPS, D), jnp.float32),
        in_specs=[pl.BlockSpec(memory_space=pltpu.VMEM)] * 7,
        out_specs=pl.BlockSpec(memory_space=pltpu.VMEM),
        scratch_shapes=[
            pltpu.VMEM((B, D), jnp.float32),
            pltpu.VMEM((2, 2, 3, S, RH, D), jnp.float32),
            pltpu.VMEM((N_DEV - 1, RPS, D), jnp.float32),
            pltpu.SemaphoreType.DMA((2, 2, 3, S)),
            pltpu.SemaphoreType.DMA((2, 2, 3, S)),
            pltpu.SemaphoreType.DMA((N_DEV - 1,)),
            pltpu.SemaphoreType.DMA((N_DEV - 1,)),
        ],
        compiler_params=pltpu.CompilerParams(collective_id=0),
    )(x, Win0, Wout0, Win1, Wout1, Win2, Wout2)


# device time: 29748 ns/iter; 1.0991x vs baseline; 1.0242x over previous
import jax
import jax.numpy as jnp
from jax import lax
from jax.experimental import pallas as pl
from jax.experimental.pallas import tpu as pltpu

N_DEV = 16
B = 128
D = 128
RPS = B // N_DEV
S = 4
RH = B // S


def kernel(x, Win0, Wout0, Win1, Wout1, Win2, Wout2):
    def body(x_ref, win0, wout0, win1, wout1, win2, wout2,
             out_ref, acc_ref, comm_ref, rs_ref,
             send_sems, recv_sems, rs_send_sems, rs_recv_sems):
        my = lax.axis_index("i")
        j = my % 4
        k = my // 4
        base = my - j

        wins = [win0, win1, win2]
        wouts = [wout0, wout1, wout2]

        barrier_sem = pltpu.get_barrier_semaphore()
        for d in (1, 2, 3):
            for peer in (base + (j + d) % 4, 4 * ((k + d) % 4) + j):
                pl.semaphore_signal(
                    barrier_sem, inc=1,
                    device_id=(peer,), device_id_type=pl.DeviceIdType.MESH,
                )

        peers = [
            lambda d: base + (j + d) % 4,
            lambda d: 4 * ((k + d) % 4) + j,
        ]
        handles = {}

        def rows(s):
            return pl.ds(s * RH, RH)

        def start_phase(r, p, s):
            lst = []
            for d in (1, 2, 3):
                rdma = pltpu.make_async_remote_copy(
                    src_ref=acc_ref.at[rows(s), :],
                    dst_ref=comm_ref.at[r, p, d - 1, s],
                    send_sem=send_sems.at[r, p, d - 1, s],
                    recv_sem=recv_sems.at[r, p, d - 1, s],
                    device_id=(peers[p](d),),
                    device_id_type=pl.DeviceIdType.MESH,
                )
                rdma.start()
                lst.append(rdma)
            handles[(r, p, s)] = lst

        def finish_phase(r, p, s):
            for rdma in handles[(r, p, s)]:
                rdma.wait()
            acc_ref[rows(s), :] = (
                acc_ref[rows(s), :]
                + comm_ref[r, p, 0, s]
                + comm_ref[r, p, 1, s]
                + comm_ref[r, p, 2, s]
            )

        def compute(r, s):
            xcur = x_ref[rows(s), :] if r == 0 else acc_ref[rows(s), :]
            h = jnp.maximum(
                jnp.dot(xcur, wins[r][:, :], preferred_element_type=jnp.float32),
                0.0,
            )
            acc_ref[rows(s), :] = jnp.dot(
                h, wouts[r][:, :], preferred_element_type=jnp.float32
            )

        compute(0, 0)
        pl.semaphore_wait(barrier_sem, 6)
        start_phase(0, 0, 0)
        for s in range(1, S):
            compute(0, s)
            start_phase(0, 0, s)

        for r in (0, 1):
            for s in range(S):
                finish_phase(r, 0, s)
                start_phase(r, 1, s)
            for s in range(S):
                finish_phase(r, 1, s)
                compute(r + 1, s)
                if r == 0:
                    start_phase(1, 0, s)

        rdmas = []
        for d in range(1, N_DEV):
            tgt = (my + d) % N_DEV
            rdma = pltpu.make_async_remote_copy(
                src_ref=acc_ref.at[pl.ds(tgt * RPS, RPS), :],
                dst_ref=rs_ref.at[d - 1],
                send_sem=rs_send_sems.at[d - 1],
                recv_sem=rs_recv_sems.at[d - 1],
                device_id=(tgt,),
                device_id_type=pl.DeviceIdType.MESH,
            )
            rdma.start()
            rdmas.append(rdma)
        for rdma in rdmas:
            rdma.wait()
        total = acc_ref[pl.ds(my * RPS, RPS), :]
        for d in range(1, N_DEV):
            total = total + rs_ref[d - 1]
        out_ref[:, :] = total

    return pl.pallas_call(
        body,
        out_shape=jax.ShapeDtypeStruct((RPS, D), jnp.float32),
        in_specs=[pl.BlockSpec(memory_space=pltpu.VMEM)] * 7,
        out_specs=pl.BlockSpec(memory_space=pltpu.VMEM),
        scratch_shapes=[
            pltpu.VMEM((B, D), jnp.float32),
            pltpu.VMEM((2, 2, 3, S, RH, D), jnp.float32),
            pltpu.VMEM((N_DEV - 1, RPS, D), jnp.float32),
            pltpu.SemaphoreType.DMA((2, 2, 3, S)),
            pltpu.SemaphoreType.DMA((2, 2, 3, S)),
            pltpu.SemaphoreType.DMA((N_DEV - 1,)),
            pltpu.SemaphoreType.DMA((N_DEV - 1,)),
        ],
        compiler_params=pltpu.CompilerParams(collective_id=0),
    )(x, Win0, Wout0, Win1, Wout1, Win2, Wout2)
